# Initial kernel scaffold; baseline (speedup 1.0000x reference)
import jax
import jax.numpy as jnp
from jax.experimental import pallas as pl


def kernel(x, edge_index, Wq, bq, Wk, bk, Wv, bv, We, be, Wea, bea, Wo, bo):
    raise NotImplementedError("write your pallas kernel here")


# speedup vs baseline: 16.6188x; 16.6188x over previous
import functools

import jax
import jax.numpy as jnp
from jax import lax
from jax.experimental import pallas as pl
from jax.experimental.pallas import tpu as pltpu
from jax.experimental.pallas import tpu_sc as plsc

HIDDEN = 128
HEADS = 8
HD = 16
NN = 10000
NE = 320000
NPAD = 10240
W = 144
WA = 136
SCALE = HD ** -0.5

NC, NS = 2, 16
B = 64
NB = 157
EPAD = NC * NS * NB * B
RPS = NPAD // NS

R = 512


def _pre_body(x_ref, wq, bq, wk, bk, wv, bv, we, weap, be2, beap,
              qa_ref, ka_ref, v_ref):
    x = x_ref[...]
    wc = jnp.dot(we[...], weap[...], preferred_element_type=jnp.float32)
    c0 = jnp.dot(be2[...], weap[...], preferred_element_type=jnp.float32) + beap[...]
    qa_ref[:, :HIDDEN] = jnp.dot(x, wq[...], preferred_element_type=jnp.float32) + bq[...]
    qa_ref[:, HIDDEN:] = jnp.dot(x, wc[HIDDEN:], preferred_element_type=jnp.float32) + c0
    ka_ref[:, :HIDDEN] = jnp.dot(x, wk[...], preferred_element_type=jnp.float32) + bk[...]
    ka_ref[:, HIDDEN:] = jnp.dot(x, wc[:HIDDEN], preferred_element_type=jnp.float32)
    v_ref[...] = jnp.dot(x, wv[...], preferred_element_type=jnp.float32) + bv[...]


def _edge_body(qa_hbm, ka_hbm, v_hbm, src_hbm, tgt_hbm, out_hbm,
               idx_s, idx_t, qa_r, ka_r, v_r, comb, acc_sh):
    cid = lax.axis_index("c")
    sid = lax.axis_index("s")
    wid = sid * NC + cid

    zv = jnp.zeros((16,), jnp.float32)

    def zrow(r, _):
        for cc in range(WA // 16):
            comb[r, pl.ds(cc * 16, 16)] = zv
        comb[r, pl.ds(WA - 16, 16)] = zv
        return 0
    lax.fori_loop(0, B, zrow, 0)

    def zchunk(i, _):
        pltpu.sync_copy(comb, acc_sh.at[pl.ds(sid * RPS + i * B, B)])
        return 0
    lax.fori_loop(0, RPS // B, zchunk, 0)

    plsc.subcore_barrier()

    lanes = lax.iota(jnp.int32, 16)
    ebase = wid * (NB * B)

    def body(it, _):
        eo = ebase + it * B
        pltpu.sync_copy(src_hbm.at[pl.ds(eo, B)], idx_s)
        pltpu.sync_copy(tgt_hbm.at[pl.ds(eo, B)], idx_t)
        pltpu.sync_copy(qa_hbm.at[idx_t], qa_r)
        pltpu.sync_copy(ka_hbm.at[idx_s], ka_r)
        pltpu.sync_copy(v_hbm.at[idx_s], v_r)
        for g in range(B // 16):
            rows = lanes + (g * 16)
            for h in range(HEADS):
                cb = jnp.full((16,), HIDDEN + h, jnp.int32)
                bias = (plsc.load_gather(qa_r, [rows, cb])
                        + plsc.load_gather(ka_r, [rows, cb]))
                acc = jnp.zeros((16,), jnp.float32)
                for d in range(HD):
                    cc = jnp.full((16,), h * HD + d, jnp.int32)
                    acc = acc + (plsc.load_gather(qa_r, [rows, cc])
                                 * plsc.load_gather(ka_r, [rows, cc]))
                s = acc * SCALE + bias
                t = jnp.exp(jnp.clip(s, -60.0, 60.0))
                plsc.store_scatter(comb, [rows, cb], t)
                for d in range(HD):
                    cc = jnp.full((16,), h * HD + d, jnp.int32)
                    vv = plsc.load_gather(v_r, [rows, cc])
                    plsc.store_scatter(comb, [rows, cc], vv * t)
        pltpu.sync_copy(comb, acc_sh.at[idx_t], add=True)
        return 0
    lax.fori_loop(0, NB, body, 0)

    plsc.subcore_barrier()

    def drain(i, _):
        r0 = sid * RPS + i * B
        pltpu.sync_copy(acc_sh.at[pl.ds(r0, B)], comb)
        pltpu.sync_copy(comb, out_hbm.at[cid, pl.ds(r0, B)])
        return 0
    lax.fori_loop(0, RPS // B, drain, 0)


_edge_call = functools.partial(
    pl.kernel,
    out_type=jax.ShapeDtypeStruct((NC, NPAD, WA), jnp.float32),
    mesh=plsc.VectorSubcoreMesh(core_axis_name="c", subcore_axis_name="s"),
    compiler_params=pltpu.CompilerParams(use_tc_tiling_on_sc=False,
                                         needs_layout_passes=False),
    scratch_types=[
        pltpu.VMEM((B,), jnp.int32),
        pltpu.VMEM((B,), jnp.int32),
        pltpu.VMEM((B, W), jnp.float32),
        pltpu.VMEM((B, W), jnp.float32),
        pltpu.VMEM((B, HIDDEN), jnp.float32),
        pltpu.VMEM((B, WA), jnp.float32),
        pltpu.VMEM_SHARED((NPAD, WA), jnp.float32),
    ],
)(_edge_body)


def _post_body(acc_ref, wo, bo, o_ref):
    a = acc_ref[0] + acc_ref[1]
    num = a[:, :HIDDEN]
    den = a[:, HIDDEN:HIDDEN + HEADS]
    recip = jnp.where(den > 0, 1.0 / den, 0.0)
    i0 = lax.broadcasted_iota(jnp.int32, (HEADS, HIDDEN), 0)
    i1 = lax.broadcasted_iota(jnp.int32, (HEADS, HIDDEN), 1)
    sel = (i1 // HD == i0).astype(jnp.float32)
    den128 = jnp.dot(recip, sel, preferred_element_type=jnp.float32)
    o_ref[...] = (jnp.dot(num * den128, wo[...],
                          preferred_element_type=jnp.float32) + bo[...])


def kernel(x, edge_index, Wq, bq, Wk, bk, Wv, bv, We, be, Wea, bea, Wo, bo):
    x2d = jnp.pad(x[0], ((0, NPAD - NN), (0, 0)))
    pad = jnp.full((EPAD - NE,), NPAD - 1, jnp.int32)
    src = jnp.concatenate([edge_index[0, :, 0], pad])
    tgt = jnp.concatenate([edge_index[0, :, 1], pad])
    weap = jnp.pad(Wea, ((0, 0), (0, HD - HEADS)))
    beap = jnp.pad(bea, (0, HD - HEADS)).reshape(1, HD)
    be2 = be.reshape(1, -1)

    full = lambda s: pl.BlockSpec(s, lambda i: (0,) * len(s))
    qa, ka, vt = pl.pallas_call(
        _pre_body,
        grid=(NPAD // R,),
        in_specs=[
            pl.BlockSpec((R, HIDDEN), lambda i: (i, 0)),
            full((HIDDEN, HIDDEN)), full((1, HIDDEN)),
            full((HIDDEN, HIDDEN)), full((1, HIDDEN)),
            full((HIDDEN, HIDDEN)), full((1, HIDDEN)),
            full((2 * HIDDEN, 64)), full((64, HD)),
            full((1, 64)), full((1, HD)),
        ],
        out_specs=[
            pl.BlockSpec((R, W), lambda i: (i, 0)),
            pl.BlockSpec((R, W), lambda i: (i, 0)),
            pl.BlockSpec((R, HIDDEN), lambda i: (i, 0)),
        ],
        out_shape=[
            jax.ShapeDtypeStruct((NPAD, W), jnp.float32),
            jax.ShapeDtypeStruct((NPAD, W), jnp.float32),
            jax.ShapeDtypeStruct((NPAD, HIDDEN), jnp.float32),
        ],
    )(x2d, Wq, bq.reshape(1, -1), Wk, bk.reshape(1, -1),
      Wv, bv.reshape(1, -1), We, weap, be2, beap)

    acc = _edge_call(qa, ka, vt, src, tgt)

    out = pl.pallas_call(
        _post_body,
        grid=(NN // 400,),
        in_specs=[
            pl.BlockSpec((NC, 400, WA), lambda i: (0, i, 0)),
            full((HIDDEN, HIDDEN)), full((1, HIDDEN)),
        ],
        out_specs=pl.BlockSpec((400, HIDDEN), lambda i: (i, 0)),
        out_shape=jax.ShapeDtypeStruct((NN, HIDDEN), jnp.float32),
    )(acc, Wo, bo.reshape(1, -1))
    return out.reshape(1, NN, HIDDEN)


# speedup vs baseline: 21.8045x; 1.3120x over previous
import functools

import jax
import jax.numpy as jnp
from jax import lax
from jax.experimental import pallas as pl
from jax.experimental.pallas import tpu as pltpu
from jax.experimental.pallas import tpu_sc as plsc

HIDDEN = 128
HEADS = 8
HD = 16
NN = 10000
NE = 320000
NPAD = 10240
W = 144
WA = 136
SCALE = HD ** -0.5

NC, NS = 2, 16
B = 16
NB = 628
EPAD = NC * NS * NB * B
RPS = NPAD // NS

R = 512


def _pre_body(x_ref, wq, bq, wk, bk, wv, bv, we, weap, be2, beap,
              qa_ref, ka_ref, v_ref):
    x = x_ref[...]
    wc = jnp.dot(we[...], weap[...], preferred_element_type=jnp.float32)
    c0 = jnp.dot(be2[...], weap[...], preferred_element_type=jnp.float32) + beap[...]
    qa_ref[:, :HIDDEN] = jnp.dot(x, wq[...], preferred_element_type=jnp.float32) + bq[...]
    qa_ref[:, HIDDEN:] = jnp.dot(x, wc[HIDDEN:], preferred_element_type=jnp.float32) + c0
    ka_ref[:, :HIDDEN] = jnp.dot(x, wk[...], preferred_element_type=jnp.float32) + bk[...]
    ka_ref[:, HIDDEN:] = jnp.dot(x, wc[:HIDDEN], preferred_element_type=jnp.float32)
    v_ref[...] = jnp.dot(x, wv[...], preferred_element_type=jnp.float32) + bv[...]


def _edge_body(qa_hbm, ka_hbm, v_hbm, src_hbm, tgt_hbm, out_hbm,
               is0, is1, is2, is3, it0, it1, it2, it3,
               qa0, qa1, ka0, ka1, v0, v1, cb0, cb1,
               ise0, ise1, ise2, ise3, gse0, gse1, sse0, sse1,
               acc_sh):
    iss = (is0, is1, is2, is3)
    its = (it0, it1, it2, it3)
    qas = (qa0, qa1)
    kas = (ka0, ka1)
    vvs = (v0, v1)
    cbs = (cb0, cb1)
    isems = (ise0, ise1, ise2, ise3)
    gsems = (gse0, gse1)
    ssems = (sse0, sse1)

    cid = lax.axis_index("c")
    sid = lax.axis_index("s")
    wid = sid * NC + cid

    zv = jnp.zeros((16,), jnp.float32)

    def zrow(r, _):
        for cc in range(WA // 16):
            cb0[r, pl.ds(cc * 16, 16)] = zv
        cb0[r, pl.ds(WA - 16, 16)] = zv
        return 0
    lax.fori_loop(0, B, zrow, 0)

    def zchunk(i, _):
        pltpu.sync_copy(cb0, acc_sh.at[pl.ds(sid * RPS + i * B, B)])
        return 0
    lax.fori_loop(0, RPS // B, zchunk, 0)

    plsc.subcore_barrier()

    lanes = lax.iota(jnp.int32, 16)
    ebase = wid * (NB * B)

    def fire_idx(q, b):
        eo = ebase + b * B
        pltpu.async_copy(src_hbm.at[pl.ds(eo, B)], iss[q], isems[q])
        pltpu.async_copy(tgt_hbm.at[pl.ds(eo, B)], its[q], isems[q])

    def wait_idx(q):
        pltpu.make_async_copy(src_hbm.at[pl.ds(0, B)], iss[q], isems[q]).wait()
        pltpu.make_async_copy(tgt_hbm.at[pl.ds(0, B)], its[q], isems[q]).wait()

    def fire_gathers(p, q):
        pltpu.async_copy(qa_hbm.at[its[q]], qas[p], gsems[p])
        pltpu.async_copy(ka_hbm.at[iss[q]], kas[p], gsems[p])
        pltpu.async_copy(v_hbm.at[iss[q]], vvs[p], gsems[p])

    def wait_gathers(p):
        pltpu.make_async_copy(qa_hbm.at[its[0]], qas[p], gsems[p]).wait()
        pltpu.make_async_copy(ka_hbm.at[iss[0]], kas[p], gsems[p]).wait()
        pltpu.make_async_copy(v_hbm.at[iss[0]], vvs[p], gsems[p]).wait()

    def wait_scatter(p):
        pltpu.make_async_copy(cbs[p], acc_sh.at[its[0]], ssems[p]).wait()

    def compute(p):
        qa_r, ka_r, v_r, comb = qas[p], kas[p], vvs[p], cbs[p]
        rows = lanes
        for h in range(HEADS):
            cb = jnp.full((16,), HIDDEN + h, jnp.int32)
            bias = (plsc.load_gather(qa_r, [rows, cb])
                    + plsc.load_gather(ka_r, [rows, cb]))
            acc = jnp.zeros((16,), jnp.float32)
            for d in range(HD):
                cc = jnp.full((16,), h * HD + d, jnp.int32)
                acc = acc + (plsc.load_gather(qa_r, [rows, cc])
                             * plsc.load_gather(ka_r, [rows, cc]))
            s = acc * SCALE + bias
            t = jnp.exp(jnp.clip(s, -60.0, 60.0))
            plsc.store_scatter(comb, [rows, cb], t)
            for d in range(HD):
                cc = jnp.full((16,), h * HD + d, jnp.int32)
                vv = plsc.load_gather(v_r, [rows, cc])
                plsc.store_scatter(comb, [rows, cc], vv * t)

    fire_idx(0, 0)
    fire_idx(1, 1)
    wait_idx(0)
    fire_gathers(0, 0)

    def body(j, _):
        for ph in range(4):
            p = ph % 2
            b = j * 4 + ph
            if ph >= 2:
                wait_scatter(p)
            else:
                @pl.when(j > 0)
                def _():
                    wait_scatter(p)

            @pl.when(b + 2 < NB)
            def _():
                fire_idx((ph + 2) % 4, b + 2)

            @pl.when(b + 1 < NB)
            def _():
                wait_idx((ph + 1) % 4)
                fire_gathers((ph + 1) % 2, (ph + 1) % 4)

            wait_gathers(p)
            compute(p)
            pltpu.async_copy(cbs[p], acc_sh.at[its[ph % 4]], ssems[p],
                             add=True)
        return 0
    lax.fori_loop(0, NB // 4, body, 0)
    wait_scatter(0)
    wait_scatter(1)

    plsc.subcore_barrier()

    def drain(i, _):
        r0 = sid * RPS + i * B
        pltpu.sync_copy(acc_sh.at[pl.ds(r0, B)], cb0)
        pltpu.sync_copy(cb0, out_hbm.at[cid, pl.ds(r0, B)])
        return 0
    lax.fori_loop(0, RPS // B, drain, 0)


_edge_call = functools.partial(
    pl.kernel,
    out_type=jax.ShapeDtypeStruct((NC, NPAD, WA), jnp.float32),
    mesh=plsc.VectorSubcoreMesh(core_axis_name="c", subcore_axis_name="s"),
    compiler_params=pltpu.CompilerParams(use_tc_tiling_on_sc=False,
                                         needs_layout_passes=False),
    scratch_types=(
        [pltpu.VMEM((B,), jnp.int32)] * 8
        + [pltpu.VMEM((B, W), jnp.float32)] * 4
        + [pltpu.VMEM((B, HIDDEN), jnp.float32)] * 2
        + [pltpu.VMEM((B, WA), jnp.float32)] * 2
        + [pltpu.SemaphoreType.DMA] * 8
        + [pltpu.VMEM_SHARED((NPAD, WA), jnp.float32)]
    ),
)(_edge_body)


def _post_body(acc_ref, wo, bo, o_ref):
    a = acc_ref[0] + acc_ref[1]
    num = a[:, :HIDDEN]
    den = a[:, HIDDEN:HIDDEN + HEADS]
    recip = jnp.where(den > 0, 1.0 / den, 0.0)
    i0 = lax.broadcasted_iota(jnp.int32, (HEADS, HIDDEN), 0)
    i1 = lax.broadcasted_iota(jnp.int32, (HEADS, HIDDEN), 1)
    sel = (i1 // HD == i0).astype(jnp.float32)
    den128 = jnp.dot(recip, sel, preferred_element_type=jnp.float32)
    o_ref[...] = (jnp.dot(num * den128, wo[...],
                          preferred_element_type=jnp.float32) + bo[...])


def kernel(x, edge_index, Wq, bq, Wk, bk, Wv, bv, We, be, Wea, bea, Wo, bo):
    x2d = jnp.pad(x[0], ((0, NPAD - NN), (0, 0)))
    pad = jnp.full((EPAD - NE,), NPAD - 1, jnp.int32)
    src = jnp.concatenate([edge_index[0, :, 0], pad])
    tgt = jnp.concatenate([edge_index[0, :, 1], pad])
    weap = jnp.pad(Wea, ((0, 0), (0, HD - HEADS)))
    beap = jnp.pad(bea, (0, HD - HEADS)).reshape(1, HD)
    be2 = be.reshape(1, -1)

    full = lambda s: pl.BlockSpec(s, lambda i: (0,) * len(s))
    qa, ka, vt = pl.pallas_call(
        _pre_body,
        grid=(NPAD // R,),
        in_specs=[
            pl.BlockSpec((R, HIDDEN), lambda i: (i, 0)),
            full((HIDDEN, HIDDEN)), full((1, HIDDEN)),
            full((HIDDEN, HIDDEN)), full((1, HIDDEN)),
            full((HIDDEN, HIDDEN)), full((1, HIDDEN)),
            full((2 * HIDDEN, 64)), full((64, HD)),
            full((1, 64)), full((1, HD)),
        ],
        out_specs=[
            pl.BlockSpec((R, W), lambda i: (i, 0)),
            pl.BlockSpec((R, W), lambda i: (i, 0)),
            pl.BlockSpec((R, HIDDEN), lambda i: (i, 0)),
        ],
        out_shape=[
            jax.ShapeDtypeStruct((NPAD, W), jnp.float32),
            jax.ShapeDtypeStruct((NPAD, W), jnp.float32),
            jax.ShapeDtypeStruct((NPAD, HIDDEN), jnp.float32),
        ],
    )(x2d, Wq, bq.reshape(1, -1), Wk, bk.reshape(1, -1),
      Wv, bv.reshape(1, -1), We, weap, be2, beap)

    acc = _edge_call(qa, ka, vt, src, tgt)

    out = pl.pallas_call(
        _post_body,
        grid=(NN // 400,),
        in_specs=[
            pl.BlockSpec((NC, 400, WA), lambda i: (0, i, 0)),
            full((HIDDEN, HIDDEN)), full((1, HIDDEN)),
        ],
        out_specs=pl.BlockSpec((400, HIDDEN), lambda i: (i, 0)),
        out_shape=jax.ShapeDtypeStruct((NN, HIDDEN), jnp.float32),
    )(acc, Wo, bo.reshape(1, -1))
    return out.reshape(1, NN, HIDDEN)


# speedup vs baseline: 21.8337x; 1.0013x over previous
import functools

import jax
import jax.numpy as jnp
from jax import lax
from jax.experimental import pallas as pl
from jax.experimental.pallas import tpu as pltpu
from jax.experimental.pallas import tpu_sc as plsc

HIDDEN = 128
HEADS = 8
HD = 16
NN = 10000
NE = 320000
NPAD = 10240
W = 144
WA = 136
SCALE = HD ** -0.5

NC, NS = 2, 16
B = 16
NB = 628
EPAD = NC * NS * NB * B
RPS = NPAD // NS

R = 512


def _pre_body(x_ref, wq, bq, wk, bk, wv, bv, we, weap, be2, beap,
              qa_ref, ka_ref, v_ref):
    x = x_ref[...]
    wc = jnp.dot(we[...], weap[...], preferred_element_type=jnp.float32)
    c0 = jnp.dot(be2[...], weap[...], preferred_element_type=jnp.float32) + beap[...]
    qa_ref[:, :HIDDEN] = jnp.dot(x, wq[...], preferred_element_type=jnp.float32) + bq[...]
    qa_ref[:, HIDDEN:] = jnp.dot(x, wc[HIDDEN:], preferred_element_type=jnp.float32) + c0
    ka_ref[:, :HIDDEN] = jnp.dot(x, wk[...], preferred_element_type=jnp.float32) + bk[...]
    ka_ref[:, HIDDEN:] = jnp.dot(x, wc[:HIDDEN], preferred_element_type=jnp.float32)
    v_ref[...] = jnp.dot(x, wv[...], preferred_element_type=jnp.float32) + bv[...]


def _edge_body(qa_hbm, ka_hbm, v_hbm, src_hbm, tgt_hbm, out_hbm,
               src_l, tgt_l, qa0, qa1, ka0, ka1, v0, v1, cb0, cb1,
               gse0, gse1, sse0, sse1,
               acc_sh):
    qas = (qa0, qa1)
    kas = (ka0, ka1)
    vvs = (v0, v1)
    cbs = (cb0, cb1)
    gsems = (gse0, gse1)
    ssems = (sse0, sse1)

    cid = lax.axis_index("c")
    sid = lax.axis_index("s")
    wid = sid * NC + cid

    zv = jnp.zeros((16,), jnp.float32)

    def zrow(r, _):
        for cc in range(WA // 16):
            cb0[r, pl.ds(cc * 16, 16)] = zv
        cb0[r, pl.ds(WA - 16, 16)] = zv
        return 0
    lax.fori_loop(0, B, zrow, 0)

    def zchunk(i, _):
        pltpu.sync_copy(cb0, acc_sh.at[pl.ds(sid * RPS + i * B, B)])
        return 0
    lax.fori_loop(0, RPS // B, zchunk, 0)

    pltpu.sync_copy(src_hbm.at[wid], src_l)
    pltpu.sync_copy(tgt_hbm.at[wid], tgt_l)

    plsc.subcore_barrier()

    lanes = lax.iota(jnp.int32, 16)

    def fire_gathers(p, b):
        pltpu.async_copy(qa_hbm.at[tgt_l.at[b, 0]], qas[p], gsems[p])
        pltpu.async_copy(ka_hbm.at[src_l.at[b, 0]], kas[p], gsems[p])
        pltpu.async_copy(v_hbm.at[src_l.at[b, 0]], vvs[p], gsems[p])

    def wait_gathers(p):
        pltpu.make_async_copy(qa_hbm.at[tgt_l.at[0, 0]], qas[p], gsems[p]).wait()
        pltpu.make_async_copy(ka_hbm.at[src_l.at[0, 0]], kas[p], gsems[p]).wait()
        pltpu.make_async_copy(v_hbm.at[src_l.at[0, 0]], vvs[p], gsems[p]).wait()

    def wait_scatter(p):
        pltpu.make_async_copy(cbs[p], acc_sh.at[tgt_l.at[0, 0]], ssems[p]).wait()

    def compute(p):
        qa_r, ka_r, v_r, comb = qas[p], kas[p], vvs[p], cbs[p]
        rows = lanes
        for h in range(HEADS):
            cb = jnp.full((16,), HIDDEN + h, jnp.int32)
            bias = (plsc.load_gather(qa_r, [rows, cb])
                    + plsc.load_gather(ka_r, [rows, cb]))
            acc = jnp.zeros((16,), jnp.float32)
            for d in range(HD):
                cc = jnp.full((16,), h * HD + d, jnp.int32)
                acc = acc + (plsc.load_gather(qa_r, [rows, cc])
                             * plsc.load_gather(ka_r, [rows, cc]))
            s = acc * SCALE + bias
            t = jnp.exp(jnp.clip(s, -60.0, 60.0))
            plsc.store_scatter(comb, [rows, cb], t)
            for d in range(HD):
                cc = jnp.full((16,), h * HD + d, jnp.int32)
                vv = plsc.load_gather(v_r, [rows, cc])
                plsc.store_scatter(comb, [rows, cc], vv * t)

    fire_gathers(0, 0)

    def body(j, _):
        for ph in range(2):
            p = ph
            b = j * 2 + ph

            @pl.when(j > 0)
            def _():
                wait_scatter(p)

            @pl.when(b + 1 < NB)
            def _():
                fire_gathers((ph + 1) % 2, b + 1)

            wait_gathers(p)
            compute(p)
            pltpu.async_copy(cbs[p], acc_sh.at[tgt_l.at[b, 0]], ssems[p],
                             add=True)
        return 0
    lax.fori_loop(0, NB // 2, body, 0)
    wait_scatter(0)
    wait_scatter(1)

    plsc.subcore_barrier()

    def drain(i, _):
        r0 = sid * RPS + i * B
        pltpu.sync_copy(acc_sh.at[pl.ds(r0, B)], cb0)
        pltpu.sync_copy(cb0, out_hbm.at[cid, pl.ds(r0, B)])
        return 0
    lax.fori_loop(0, RPS // B, drain, 0)


_edge_call = functools.partial(
    pl.kernel,
    out_type=jax.ShapeDtypeStruct((NC, NPAD, WA), jnp.float32),
    mesh=plsc.VectorSubcoreMesh(core_axis_name="c", subcore_axis_name="s"),
    compiler_params=pltpu.CompilerParams(use_tc_tiling_on_sc=False,
                                         needs_layout_passes=False),
    scratch_types=(
        [pltpu.VMEM((NB, 1, B), jnp.int32)] * 2
        + [pltpu.VMEM((B, W), jnp.float32)] * 4
        + [pltpu.VMEM((B, HIDDEN), jnp.float32)] * 2
        + [pltpu.VMEM((B, WA), jnp.float32)] * 2
        + [pltpu.SemaphoreType.DMA] * 4
        + [pltpu.VMEM_SHARED((NPAD, WA), jnp.float32)]
    ),
)(_edge_body)


def _post_body(acc_ref, wo, bo, o_ref):
    a = acc_ref[0] + acc_ref[1]
    num = a[:, :HIDDEN]
    den = a[:, HIDDEN:HIDDEN + HEADS]
    recip = jnp.where(den > 0, 1.0 / den, 0.0)
    i0 = lax.broadcasted_iota(jnp.int32, (HEADS, HIDDEN), 0)
    i1 = lax.broadcasted_iota(jnp.int32, (HEADS, HIDDEN), 1)
    sel = (i1 // HD == i0).astype(jnp.float32)
    den128 = jnp.dot(recip, sel, preferred_element_type=jnp.float32)
    o_ref[...] = (jnp.dot(num * den128, wo[...],
                          preferred_element_type=jnp.float32) + bo[...])


def kernel(x, edge_index, Wq, bq, Wk, bk, Wv, bv, We, be, Wea, bea, Wo, bo):
    x2d = jnp.pad(x[0], ((0, NPAD - NN), (0, 0)))
    pad = jnp.full((EPAD - NE,), NPAD - 1, jnp.int32)
    src = jnp.concatenate([edge_index[0, :, 0], pad]).reshape(NC * NS, NB, 1, B)
    tgt = jnp.concatenate([edge_index[0, :, 1], pad]).reshape(NC * NS, NB, 1, B)
    weap = jnp.pad(Wea, ((0, 0), (0, HD - HEADS)))
    beap = jnp.pad(bea, (0, HD - HEADS)).reshape(1, HD)
    be2 = be.reshape(1, -1)

    full = lambda s: pl.BlockSpec(s, lambda i: (0,) * len(s))
    qa, ka, vt = pl.pallas_call(
        _pre_body,
        grid=(NPAD // R,),
        in_specs=[
            pl.BlockSpec((R, HIDDEN), lambda i: (i, 0)),
            full((HIDDEN, HIDDEN)), full((1, HIDDEN)),
            full((HIDDEN, HIDDEN)), full((1, HIDDEN)),
            full((HIDDEN, HIDDEN)), full((1, HIDDEN)),
            full((2 * HIDDEN, 64)), full((64, HD)),
            full((1, 64)), full((1, HD)),
        ],
        out_specs=[
            pl.BlockSpec((R, W), lambda i: (i, 0)),
            pl.BlockSpec((R, W), lambda i: (i, 0)),
            pl.BlockSpec((R, HIDDEN), lambda i: (i, 0)),
        ],
        out_shape=[
            jax.ShapeDtypeStruct((NPAD, W), jnp.float32),
            jax.ShapeDtypeStruct((NPAD, W), jnp.float32),
            jax.ShapeDtypeStruct((NPAD, HIDDEN), jnp.float32),
        ],
    )(x2d, Wq, bq.reshape(1, -1), Wk, bk.reshape(1, -1),
      Wv, bv.reshape(1, -1), We, weap, be2, beap)

    acc = _edge_call(qa, ka, vt, src, tgt)

    out = pl.pallas_call(
        _post_body,
        grid=(NN // 400,),
        in_specs=[
            pl.BlockSpec((NC, 400, WA), lambda i: (0, i, 0)),
            full((HIDDEN, HIDDEN)), full((1, HIDDEN)),
        ],
        out_specs=pl.BlockSpec((400, HIDDEN), lambda i: (i, 0)),
        out_shape=jax.ShapeDtypeStruct((NN, HIDDEN), jnp.float32),
    )(acc, Wo, bo.reshape(1, -1))
    return out.reshape(1, NN, HIDDEN)


# speedup vs baseline: 30.0638x; 1.3769x over previous
import functools

import jax
import jax.numpy as jnp
from jax import lax
from jax.experimental import pallas as pl
from jax.experimental.pallas import tpu as pltpu
from jax.experimental.pallas import tpu_sc as plsc

HIDDEN = 128
HEADS = 8
HD = 16
NN = 10000
NE = 320000
NPAD = 10240
WQ = 144
WK = 272
VOFF = 144
WA = 136
SCALE = HD ** -0.5

NC, NS = 2, 16
B = 32
NB = 316
EPAD = NC * NS * NB * B
RPS = NPAD // NS

R = 512


def _pre_body(x_ref, wq, bq, wk, bk, wv, bv, we, weap, be2, beap,
              qa_ref, kv_ref):
    x = x_ref[...]
    wc = jnp.dot(we[...], weap[...], preferred_element_type=jnp.float32)
    c0 = jnp.dot(be2[...], weap[...], preferred_element_type=jnp.float32) + beap[...]
    qa_ref[:, :HIDDEN] = jnp.dot(x, wq[...], preferred_element_type=jnp.float32) + bq[...]
    qa_ref[:, HIDDEN:] = jnp.dot(x, wc[HIDDEN:], preferred_element_type=jnp.float32) + c0
    kv_ref[:, :HIDDEN] = jnp.dot(x, wk[...], preferred_element_type=jnp.float32) + bk[...]
    kv_ref[:, HIDDEN:VOFF] = jnp.dot(x, wc[:HIDDEN], preferred_element_type=jnp.float32)
    kv_ref[:, VOFF:] = jnp.dot(x, wv[...], preferred_element_type=jnp.float32) + bv[...]


def _edge_body(qa_hbm, kv_hbm, src_hbm, tgt_hbm, out_hbm,
               s0, s1, s2, s3, t0, t1, t2, t3,
               qa0, qa1, kv0, kv1, cb0, cb1,
               ise0, ise1, ise2, ise3, gse0, gse1, sse0, sse1,
               acc_sh):
    srcs = (s0, s1, s2, s3)
    tgts = (t0, t1, t2, t3)
    qas = (qa0, qa1)
    kvs = (kv0, kv1)
    cbs = (cb0, cb1)
    isems = (ise0, ise1, ise2, ise3)
    gsems = (gse0, gse1)
    ssems = (sse0, sse1)

    cid = lax.axis_index("c")
    sid = lax.axis_index("s")
    wid = sid * NC + cid

    zv = jnp.zeros((16,), jnp.float32)

    def zrow(r, _):
        for cc in range(WA // 16):
            cb0[r, pl.ds(cc * 16, 16)] = zv
        cb0[r, pl.ds(WA - 16, 16)] = zv
        return 0
    lax.fori_loop(0, B, zrow, 0)

    def zchunk(i, _):
        pltpu.sync_copy(cb0, acc_sh.at[pl.ds(sid * RPS + i * B, B)])
        return 0
    lax.fori_loop(0, RPS // B, zchunk, 0)
    plsc.subcore_barrier()

    lanes = lax.iota(jnp.int32, 16)

    def fire_idx(q, b):
        pltpu.async_copy(src_hbm.at[pl.ds((wid * NB + b) * B, B)], srcs[q], isems[q])
        pltpu.async_copy(tgt_hbm.at[pl.ds((wid * NB + b) * B, B)], tgts[q], isems[q])

    def wait_idx(q):
        pltpu.make_async_copy(src_hbm.at[pl.ds(0, B)], srcs[q], isems[q]).wait()
        pltpu.make_async_copy(tgt_hbm.at[pl.ds(0, B)], tgts[q], isems[q]).wait()

    def fire_gathers(p, q):
        pltpu.async_copy(qa_hbm.at[tgts[q]], qas[p], gsems[p])
        pltpu.async_copy(kv_hbm.at[srcs[q]], kvs[p], gsems[p])

    def wait_gathers(p):
        pltpu.make_async_copy(qa_hbm.at[tgts[0]], qas[p], gsems[p]).wait()
        pltpu.make_async_copy(kv_hbm.at[srcs[0]], kvs[p], gsems[p]).wait()

    def wait_scatter(p):
        pltpu.make_async_copy(cbs[p], acc_sh.at[tgts[0]], ssems[p]).wait()

    def compute(p):
        qa_r, kv_r, comb = qas[p], kvs[p], cbs[p]

        def group(g, _):
            rows = lanes + g * 16
            for h in range(HEADS):
                cb = jnp.full((16,), HIDDEN + h, jnp.int32)
                bias = (plsc.load_gather(qa_r, [rows, cb])
                        + plsc.load_gather(kv_r, [rows, cb]))
                acc = jnp.zeros((16,), jnp.float32)
                for d in range(HD):
                    cc = jnp.full((16,), h * HD + d, jnp.int32)
                    acc = acc + (plsc.load_gather(qa_r, [rows, cc])
                                 * plsc.load_gather(kv_r, [rows, cc]))
                s = acc * SCALE + bias
                t = jnp.exp(jnp.clip(s, -60.0, 60.0))
                plsc.store_scatter(comb, [rows, cb], t)
                for d in range(HD):
                    cv = jnp.full((16,), VOFF + h * HD + d, jnp.int32)
                    cc = jnp.full((16,), h * HD + d, jnp.int32)
                    vv = plsc.load_gather(kv_r, [rows, cv])
                    plsc.store_scatter(comb, [rows, cc], vv * t)
            return 0
        lax.fori_loop(0, B // 16, group, 0)

    fire_idx(0, 0)
    fire_idx(1, 1)
    wait_idx(0)
    fire_gathers(0, 0)

    def body(j, _):
        for ph in range(4):
            p = ph % 2
            b = j * 4 + ph

            if ph >= 2:
                wait_scatter(p)
            else:
                @pl.when(j > 0)
                def _():
                    wait_scatter(p)

            @pl.when(b + 2 < NB)
            def _():
                fire_idx((ph + 2) % 4, b + 2)

            @pl.when(b + 1 < NB)
            def _():
                wait_idx((ph + 1) % 4)
                fire_gathers((ph + 1) % 2, (ph + 1) % 4)

            wait_gathers(p)
            compute(p)
            pltpu.async_copy(cbs[p], acc_sh.at[tgts[ph]], ssems[p],
                             add=True)
        return 0
    lax.fori_loop(0, NB // 4, body, 0)
    wait_scatter(0)
    wait_scatter(1)

    plsc.subcore_barrier()

    def drain(i, _):
        r0 = sid * RPS + i * B
        pltpu.sync_copy(acc_sh.at[pl.ds(r0, B)], cb0)
        pltpu.sync_copy(cb0, out_hbm.at[cid, pl.ds(r0, B)])
        return 0
    lax.fori_loop(0, RPS // B, drain, 0)


_edge_call = functools.partial(
    pl.kernel,
    out_type=jax.ShapeDtypeStruct((NC, NPAD, WA), jnp.float32),
    mesh=plsc.VectorSubcoreMesh(core_axis_name="c", subcore_axis_name="s"),
    compiler_params=pltpu.CompilerParams(use_tc_tiling_on_sc=False,
                                         needs_layout_passes=False),
    scratch_types=(
        [pltpu.VMEM((B,), jnp.int32)] * 8
        + [pltpu.VMEM((B, WQ), jnp.float32)] * 2
        + [pltpu.VMEM((B, WK), jnp.float32)] * 2
        + [pltpu.VMEM((B, WA), jnp.float32)] * 2
        + [pltpu.SemaphoreType.DMA] * 8
        + [pltpu.VMEM_SHARED((NPAD, WA), jnp.float32)]
    ),
)(_edge_body)


def _post_body(acc_ref, wo, bo, o_ref):
    a = acc_ref[0] + acc_ref[1]
    num = a[:, :HIDDEN]
    den = a[:, HIDDEN:HIDDEN + HEADS]
    recip = jnp.where(den > 0, 1.0 / den, 0.0)
    i0 = lax.broadcasted_iota(jnp.int32, (HEADS, HIDDEN), 0)
    i1 = lax.broadcasted_iota(jnp.int32, (HEADS, HIDDEN), 1)
    sel = (i1 // HD == i0).astype(jnp.float32)
    den128 = jnp.dot(recip, sel, preferred_element_type=jnp.float32)
    o_ref[...] = (jnp.dot(num * den128, wo[...],
                          preferred_element_type=jnp.float32) + bo[...])


def kernel(x, edge_index, Wq, bq, Wk, bk, Wv, bv, We, be, Wea, bea, Wo, bo):
    x2d = jnp.pad(x[0], ((0, NPAD - NN), (0, 0)))
    pad = jnp.full((EPAD - NE,), NPAD - 1, jnp.int32)
    src = jnp.concatenate([edge_index[0, :, 0], pad])
    tgt = jnp.concatenate([edge_index[0, :, 1], pad])
    weap = jnp.pad(Wea, ((0, 0), (0, HD - HEADS)))
    beap = jnp.pad(bea, (0, HD - HEADS)).reshape(1, HD)
    be2 = be.reshape(1, -1)

    full = lambda s: pl.BlockSpec(s, lambda i: (0,) * len(s))
    qa, kv = pl.pallas_call(
        _pre_body,
        grid=(NPAD // R,),
        in_specs=[
            pl.BlockSpec((R, HIDDEN), lambda i: (i, 0)),
            full((HIDDEN, HIDDEN)), full((1, HIDDEN)),
            full((HIDDEN, HIDDEN)), full((1, HIDDEN)),
            full((HIDDEN, HIDDEN)), full((1, HIDDEN)),
            full((2 * HIDDEN, 64)), full((64, HD)),
            full((1, 64)), full((1, HD)),
        ],
        out_specs=[
            pl.BlockSpec((R, WQ), lambda i: (i, 0)),
            pl.BlockSpec((R, WK), lambda i: (i, 0)),
        ],
        out_shape=[
            jax.ShapeDtypeStruct((NPAD, WQ), jnp.float32),
            jax.ShapeDtypeStruct((NPAD, WK), jnp.float32),
        ],
    )(x2d, Wq, bq.reshape(1, -1), Wk, bk.reshape(1, -1),
      Wv, bv.reshape(1, -1), We, weap, be2, beap)

    acc = _edge_call(qa, kv, src, tgt)

    out = pl.pallas_call(
        _post_body,
        grid=(NN // 400,),
        in_specs=[
            pl.BlockSpec((NC, 400, WA), lambda i: (0, i, 0)),
            full((HIDDEN, HIDDEN)), full((1, HIDDEN)),
        ],
        out_specs=pl.BlockSpec((400, HIDDEN), lambda i: (i, 0)),
        out_shape=jax.ShapeDtypeStruct((NN, HIDDEN), jnp.float32),
    )(acc, Wo, bo.reshape(1, -1))
    return out.reshape(1, NN, HIDDEN)
